# baseline (speedup 1.0000x reference)
import functools

import jax
import jax.numpy as jnp
from jax import lax
from jax.experimental import pallas as pl
from jax.experimental.pallas import tpu as pltpu
from jax.experimental.pallas import tpu_sc as plsc

M = 1000000
N = 1000000
K = 16
B = 16384

_NC = 2
_NS = 16
_NW = _NC * _NS
_BPW = B // _NW
_CH = 16
_NCHUNK = _BPW // _CH


def _funk_body(uid, iid, uembT, ubias, iembT, ibias, bias128, outT,
               uidx_v, iidx_v, wua, wub, wia, wib, ub_v, ib_v, bias_v, out_t,
               sema, semb):
  wid = lax.axis_index("s") * _NC + lax.axis_index("c")
  base = wid * _BPW

  pltpu.sync_copy(uid.at[pl.ds(base, _BPW)], uidx_v)
  pltpu.sync_copy(iid.at[pl.ds(base, _BPW)], iidx_v)
  pltpu.sync_copy(bias128, bias_v)

  bcp1 = pltpu.async_copy(ubias.at[uidx_v], ub_v, sema)
  bcp2 = pltpu.async_copy(ibias.at[iidx_v], ib_v, sema)
  bcp1.wait()
  bcp2.wait()

  bvec = bias_v[pl.ds(0, 16)]
  lanes = lax.iota(jnp.int32, 16)

  for h in range(2):
    row0 = h * 8

    def issue(c, wu, wi, sem):
      s0 = c * _CH
      jw_u = lax.shift_right_logical(uidx_v[pl.ds(s0, 16)], 7)
      jw_i = lax.shift_right_logical(iidx_v[pl.ds(s0, 16)], 7)
      for lane in range(_CH):
        pltpu.async_copy(
            uembT.at[pl.ds(row0, 8), pl.ds(jw_u[lane] * 128, 128)],
            wu.at[lane], sem)
        pltpu.async_copy(
            iembT.at[pl.ds(row0, 8), pl.ds(jw_i[lane] * 128, 128)],
            wi.at[lane], sem)

    def drain(wu, wi, sem):
      for lane in range(_CH):
        pltpu.make_async_copy(uembT.at[pl.ds(0, 8), pl.ds(0, 128)],
                              wu.at[lane], sem).wait()
        pltpu.make_async_copy(iembT.at[pl.ds(0, 8), pl.ds(0, 128)],
                              wi.at[lane], sem).wait()

    def extract(c, wu, wi):
      s0 = c * _CH
      uidx16 = uidx_v[pl.ds(s0, 16)]
      iidx16 = iidx_v[pl.ds(s0, 16)]
      r_u = lax.bitwise_and(uidx16, 127)
      r_i = lax.bitwise_and(iidx16, 127)
      ub16 = ub_v[pl.ds(s0, 16)]
      ib16 = ib_v[pl.ds(s0, 16)]
      for ks in range(8):
        s = jnp.full((16,), ks, jnp.int32)
        ue_k = plsc.load_gather(wu, [lanes, s, r_u])
        ie_k = plsc.load_gather(wi, [lanes, s, r_i])
        out_t[row0 + ks, pl.ds(s0, 16)] = (
            (ue_k + ub16) * (ie_k + ib16) + bvec)

    issue(0, wua, wia, sema)

    def pair(i, carry):
      c0 = i * 2
      issue(jnp.minimum(c0 + 1, _NCHUNK - 1), wub, wib, semb)
      drain(wua, wia, sema)
      extract(c0, wua, wia)
      issue(jnp.minimum(c0 + 2, _NCHUNK - 1), wua, wia, sema)
      drain(wub, wib, semb)
      extract(c0 + 1, wub, wib)
      return carry

    lax.fori_loop(0, _NCHUNK // 2, pair, 0)

    drain(wua, wia, sema)

  pltpu.sync_copy(out_t, outT.at[:, pl.ds(base, _BPW)])


@functools.partial(
    pl.kernel,
    out_type=jax.ShapeDtypeStruct((K, B), jnp.float32),
    mesh=plsc.VectorSubcoreMesh(core_axis_name="c", subcore_axis_name="s"),
    scratch_types=[
        pltpu.VMEM((_BPW,), jnp.int32),
        pltpu.VMEM((_BPW,), jnp.int32),
        pltpu.VMEM((_CH, 8, 128), jnp.float32),
        pltpu.VMEM((_CH, 8, 128), jnp.float32),
        pltpu.VMEM((_CH, 8, 128), jnp.float32),
        pltpu.VMEM((_CH, 8, 128), jnp.float32),
        pltpu.VMEM((_BPW,), jnp.float32),
        pltpu.VMEM((_BPW,), jnp.float32),
        pltpu.VMEM((128,), jnp.float32),
        pltpu.VMEM((K, _BPW), jnp.float32),
        pltpu.SemaphoreType.DMA,
        pltpu.SemaphoreType.DMA,
    ],
    compiler_params=pltpu.CompilerParams(needs_layout_passes=False),
)
def _funk(*args):
  _funk_body(*args)


def kernel(user_id, item_id, user_emb, user_bias, item_emb, item_bias, bias):
  bias128 = jnp.broadcast_to(bias.astype(jnp.float32), (128,))
  outT = _funk(user_id.astype(jnp.int32), item_id.astype(jnp.int32),
               user_emb.T, user_bias, item_emb.T, item_bias, bias128)
  return outT.T


# speedup vs baseline: 1.0403x; 1.0403x over previous
import functools

import jax
import jax.numpy as jnp
from jax import lax
from jax.experimental import pallas as pl
from jax.experimental.pallas import tpu as pltpu
from jax.experimental.pallas import tpu_sc as plsc

M = 1000000
N = 1000000
K = 16
B = 16384

_NC = 2
_NS = 16
_NW = _NC * _NS
_BPW = B // _NW
_CH = 16


def _funk_body(uid, iid, uembT, ubias, iembT, ibias, bias128, outT,
               uidx_v, iidx_v, wu_v, wi_v, ub_v, ib_v, bias_v, out_t, sem):
  wid = lax.axis_index("s") * _NC + lax.axis_index("c")
  base = wid * _BPW

  pltpu.sync_copy(uid.at[pl.ds(base, _BPW)], uidx_v)
  pltpu.sync_copy(iid.at[pl.ds(base, _BPW)], iidx_v)
  pltpu.sync_copy(bias128, bias_v)

  bcp1 = pltpu.async_copy(ubias.at[uidx_v], ub_v, sem)
  bcp2 = pltpu.async_copy(ibias.at[iidx_v], ib_v, sem)
  bcp1.wait()
  bcp2.wait()

  bvec = bias_v[pl.ds(0, 16)]
  lanes = lax.iota(jnp.int32, 16)

  def chunk(c, carry):
    s0 = c * _CH
    uidx16 = uidx_v[pl.ds(s0, 16)]
    iidx16 = iidx_v[pl.ds(s0, 16)]
    jw_u = lax.shift_right_logical(uidx16, 7)
    jw_i = lax.shift_right_logical(iidx16, 7)
    r_u = lax.bitwise_and(uidx16, 127)
    r_i = lax.bitwise_and(iidx16, 127)

    cps = []
    for lane in range(_CH):
      cu = jw_u[lane] * 128
      ci = jw_i[lane] * 128
      cps.append(pltpu.async_copy(
          uembT.at[:, pl.ds(cu, 128)], wu_v.at[lane], sem))
      cps.append(pltpu.async_copy(
          iembT.at[:, pl.ds(ci, 128)], wi_v.at[lane], sem))
    for cp in cps:
      cp.wait()

    ub16 = ub_v[pl.ds(s0, 16)]
    ib16 = ib_v[pl.ds(s0, 16)]
    for k in range(K):
      s = jnp.full((16,), k, jnp.int32)
      ue_k = plsc.load_gather(wu_v, [lanes, s, r_u])
      ie_k = plsc.load_gather(wi_v, [lanes, s, r_i])
      out_t[k, pl.ds(s0, 16)] = (ue_k + ub16) * (ie_k + ib16) + bvec
    return carry

  lax.fori_loop(0, _BPW // _CH, chunk, 0)

  pltpu.sync_copy(out_t, outT.at[:, pl.ds(base, _BPW)])


@functools.partial(
    pl.kernel,
    out_type=jax.ShapeDtypeStruct((K, B), jnp.float32),
    mesh=plsc.VectorSubcoreMesh(core_axis_name="c", subcore_axis_name="s"),
    scratch_types=[
        pltpu.VMEM((_BPW,), jnp.int32),
        pltpu.VMEM((_BPW,), jnp.int32),
        pltpu.VMEM((_CH, 16, 128), jnp.float32),
        pltpu.VMEM((_CH, 16, 128), jnp.float32),
        pltpu.VMEM((_BPW,), jnp.float32),
        pltpu.VMEM((_BPW,), jnp.float32),
        pltpu.VMEM((128,), jnp.float32),
        pltpu.VMEM((K, _BPW), jnp.float32),
        pltpu.SemaphoreType.DMA,
    ],
    compiler_params=pltpu.CompilerParams(needs_layout_passes=False),
)
def _funk(*args):
  _funk_body(*args)


def kernel(user_id, item_id, user_emb, user_bias, item_emb, item_bias, bias):
  bias128 = jnp.broadcast_to(bias.astype(jnp.float32), (128,))
  outT = _funk(user_id.astype(jnp.int32), item_id.astype(jnp.int32),
               user_emb.T, user_bias, item_emb.T, item_bias, bias128)
  return outT.T
